# baseline (speedup 1.0000x reference)
import functools

import jax
import jax.numpy as jnp
from jax import lax
from jax.experimental import pallas as pl
from jax.experimental.pallas import tpu as pltpu
from jax.experimental.pallas import tpu_sc as plsc

TOP_K = 8
LANES = 16


def _scores_body(gw_ref, x_ref, b_ref, out_ref):
    logits = lax.dot_general(
        gw_ref[...], x_ref[...],
        dimension_numbers=(((1,), (1,)), ((), ())),
        preferred_element_type=jnp.float32,
    )
    out_ref[...] = jax.nn.sigmoid(logits + b_ref[...])


def _make_topk_sc(num_tokens, num_experts, k):
    info = plsc.get_sparse_core_info()
    num_workers = info.num_cores * info.num_subcores
    cpw = num_tokens // num_workers
    mesh = plsc.VectorSubcoreMesh(core_axis_name="c", subcore_axis_name="s")

    @functools.partial(
        pl.kernel,
        out_type=[
            jax.ShapeDtypeStruct((num_tokens * k,), jnp.float32),
            jax.ShapeDtypeStruct((num_tokens * k,), jnp.int32),
        ],
        mesh=mesh,
        scratch_types=[
            pltpu.VMEM((num_experts, cpw), jnp.float32),
            pltpu.VMEM((cpw * k,), jnp.float32),
            pltpu.VMEM((cpw * k,), jnp.int32),
        ],
        compiler_params=pltpu.CompilerParams(needs_layout_passes=False),
    )
    def topk_sc(scores_hbm, w_hbm, i_hbm, chunk, wv, iv):
        wid = lax.axis_index("s") * info.num_cores + lax.axis_index("c")
        base = wid * cpw
        pltpu.sync_copy(scores_hbm.at[:, pl.ds(base, cpw)], chunk)

        lane = jnp.arange(LANES, dtype=jnp.int32)

        def group_body(g, carry):
            col = g * LANES
            neg = jnp.full((LANES,), -1.0, jnp.float32)
            zero_i = jnp.zeros((LANES,), jnp.int32)

            def expert_body(e, st):
                ws, idxs = st
                v = chunk[e, pl.ds(col, LANES)]
                vi = jnp.full((LANES,), e, jnp.int32)
                new_ws, new_idxs = [], []
                for j in range(k):
                    gt = v > ws[j]
                    new_ws.append(jnp.where(gt, v, ws[j]))
                    new_idxs.append(jnp.where(gt, vi, idxs[j]))
                    v = jnp.where(gt, ws[j], v)
                    vi = jnp.where(gt, idxs[j], vi)
                return tuple(new_ws), tuple(new_idxs)

            ws, idxs = lax.fori_loop(
                0, num_experts, expert_body,
                (tuple([neg] * k), tuple([zero_i] * k)))

            total = ws[0]
            for j in range(1, k):
                total = total + ws[j]
            inv = 1.0 / total
            flat = (col + lane) * k
            for j in range(k):
                plsc.store_scatter(wv, [flat + j], ws[j] * inv)
                plsc.store_scatter(iv, [flat + j], idxs[j])
            return carry

        lax.fori_loop(0, cpw // LANES, group_body, 0)
        pltpu.sync_copy(wv, w_hbm.at[pl.ds(base * k, cpw * k)])
        pltpu.sync_copy(iv, i_hbm.at[pl.ds(base * k, cpw * k)])

    return topk_sc


@jax.jit
def kernel(x, gate_w, expert_bias):
    num_tokens, dim = x.shape
    num_experts = gate_w.shape[0]

    block_t = 512
    scores_t = pl.pallas_call(
        _scores_body,
        grid=(num_tokens // block_t,),
        in_specs=[
            pl.BlockSpec((num_experts, dim), lambda i: (0, 0)),
            pl.BlockSpec((block_t, dim), lambda i: (i, 0)),
            pl.BlockSpec((num_experts, 1), lambda i: (0, 0)),
        ],
        out_specs=pl.BlockSpec((num_experts, block_t), lambda i: (0, i)),
        out_shape=jax.ShapeDtypeStruct((num_experts, num_tokens), jnp.float32),
    )(gate_w, x, expert_bias.reshape(num_experts, 1))

    w_flat, i_flat = _make_topk_sc(num_tokens, num_experts, TOP_K)(scores_t)
    return w_flat.reshape(num_tokens, TOP_K), i_flat.reshape(num_tokens, TOP_K)


# speedup vs baseline: 1.0089x; 1.0089x over previous
import functools

import jax
import jax.numpy as jnp
from jax import lax
from jax.experimental import pallas as pl
from jax.experimental.pallas import tpu as pltpu
from jax.experimental.pallas import tpu_sc as plsc

TOP_K = 8
LANES = 16


def _scores_body(gw_ref, x_ref, b_ref, out_ref):
    logits = lax.dot_general(
        gw_ref[...], x_ref[...],
        dimension_numbers=(((1,), (1,)), ((), ())),
        preferred_element_type=jnp.float32,
    )
    out_ref[...] = jax.nn.sigmoid(logits + b_ref[...])


def _make_topk_sc(num_tokens, num_experts, k):
    info = plsc.get_sparse_core_info()
    num_workers = info.num_cores * info.num_subcores
    cpw = num_tokens // num_workers
    mesh = plsc.VectorSubcoreMesh(core_axis_name="c", subcore_axis_name="s")

    @functools.partial(
        pl.kernel,
        out_type=[
            jax.ShapeDtypeStruct((num_tokens * k,), jnp.float32),
            jax.ShapeDtypeStruct((num_tokens * k,), jnp.int32),
        ],
        mesh=mesh,
        scratch_types=[
            pltpu.VMEM((num_experts, cpw), jnp.float32),
            pltpu.VMEM((cpw * k,), jnp.float32),
            pltpu.VMEM((cpw * k,), jnp.int32),
        ],
        compiler_params=pltpu.CompilerParams(needs_layout_passes=False),
    )
    def topk_sc(scores_hbm, w_hbm, i_hbm, chunk, wv, iv):
        wid = lax.axis_index("s") * info.num_cores + lax.axis_index("c")
        base = wid * cpw
        pltpu.sync_copy(scores_hbm.at[:, pl.ds(base, cpw)], chunk)

        lane = jnp.arange(LANES, dtype=jnp.int32)

        def group_body(g, carry):
            col = g * LANES
            neg = jnp.full((LANES,), -1.0, jnp.float32)
            zero_i = jnp.zeros((LANES,), jnp.int32)

            def expert_body(e, st):
                ws, idxs = st
                v = chunk[e, pl.ds(col, LANES)]
                vi = jnp.full((LANES,), e, jnp.int32)
                new_ws, new_idxs = [], []
                for j in range(k):
                    gt = v > ws[j]
                    new_ws.append(jnp.where(gt, v, ws[j]))
                    new_idxs.append(jnp.where(gt, vi, idxs[j]))
                    v = jnp.where(gt, ws[j], v)
                    vi = jnp.where(gt, idxs[j], vi)
                return tuple(new_ws), tuple(new_idxs)

            ws, idxs = lax.fori_loop(
                0, num_experts, expert_body,
                (tuple([neg] * k), tuple([zero_i] * k)))

            total = ws[0]
            for j in range(1, k):
                total = total + ws[j]
            inv = 1.0 / total
            flat = (col + lane) * k
            for j in range(k):
                plsc.store_scatter(wv, [flat + j], ws[j] * inv)
                plsc.store_scatter(iv, [flat + j], idxs[j])
            return carry

        lax.fori_loop(0, cpw // LANES, group_body, 0)
        pltpu.sync_copy(wv, w_hbm.at[pl.ds(base * k, cpw * k)])
        pltpu.sync_copy(iv, i_hbm.at[pl.ds(base * k, cpw * k)])

    return topk_sc


@jax.jit
def kernel(x, gate_w, expert_bias):
    num_tokens, dim = x.shape
    num_experts = gate_w.shape[0]

    block_t = 2048
    scores_t = pl.pallas_call(
        _scores_body,
        grid=(num_tokens // block_t,),
        in_specs=[
            pl.BlockSpec((num_experts, dim), lambda i: (0, 0)),
            pl.BlockSpec((block_t, dim), lambda i: (i, 0)),
            pl.BlockSpec((num_experts, 1), lambda i: (0, 0)),
        ],
        out_specs=pl.BlockSpec((num_experts, block_t), lambda i: (0, i)),
        out_shape=jax.ShapeDtypeStruct((num_experts, num_tokens), jnp.float32),
    )(gate_w, x, expert_bias.reshape(num_experts, 1))

    w_flat, i_flat = _make_topk_sc(num_tokens, num_experts, TOP_K)(scores_t)
    return w_flat.reshape(num_tokens, TOP_K), i_flat.reshape(num_tokens, TOP_K)


# speedup vs baseline: 1.1006x; 1.0909x over previous
import functools

import jax
import jax.numpy as jnp
from jax import lax
from jax.experimental import pallas as pl
from jax.experimental.pallas import tpu as pltpu
from jax.experimental.pallas import tpu_sc as plsc

TOP_K = 8
LANES = 16
CHUNKS = 2
BLOCK_T = 1024


def _scores_body(gw_ref, x_ref, b_ref, out_ref):
    logits = lax.dot_general(
        gw_ref[...], x_ref[...],
        dimension_numbers=(((1,), (1,)), ((), ())),
        preferred_element_type=jnp.float32,
    )
    out_ref[...] = jax.nn.sigmoid(logits + b_ref[...])


def _make_topk_sc(num_tokens, num_experts, k):
    info = plsc.get_sparse_core_info()
    num_workers = info.num_cores * info.num_subcores
    cpw = num_tokens // num_workers
    mesh = plsc.VectorSubcoreMesh(core_axis_name="c", subcore_axis_name="s")
    emask = num_experts - 1
    hi_mask = ~emask

    @functools.partial(
        pl.kernel,
        out_type=[
            jax.ShapeDtypeStruct((num_tokens * k,), jnp.float32),
            jax.ShapeDtypeStruct((num_tokens * k,), jnp.int32),
        ],
        mesh=mesh,
        scratch_types=[
            pltpu.VMEM((num_experts, cpw), jnp.float32),
            pltpu.VMEM((cpw * k,), jnp.float32),
            pltpu.VMEM((cpw * k,), jnp.int32),
        ],
        compiler_params=pltpu.CompilerParams(needs_layout_passes=False),
    )
    def topk_sc(scores_hbm, w_hbm, i_hbm, chunk, wv, iv):
        wid = lax.axis_index("s") * info.num_cores + lax.axis_index("c")
        base = wid * cpw
        pltpu.sync_copy(scores_hbm.at[:, pl.ds(base, cpw)], chunk)

        lane = jnp.arange(LANES, dtype=jnp.int32)

        def group_body(g, carry):
            col = g * LANES
            neg = jnp.full((LANES,), -1.0, jnp.float32)
            zero_i = jnp.zeros((LANES,), jnp.int32)

            def expert_body(e, st):
                ws, idxs = st
                v = chunk[e, pl.ds(col, LANES)]
                vi = jnp.full((LANES,), e, jnp.int32)
                new_ws, new_idxs = [], []
                for j in range(k):
                    gt = v > ws[j]
                    new_ws.append(jnp.where(gt, v, ws[j]))
                    new_idxs.append(jnp.where(gt, vi, idxs[j]))
                    v = jnp.where(gt, ws[j], v)
                    vi = jnp.where(gt, idxs[j], vi)
                return tuple(new_ws), tuple(new_idxs)

            ws, idxs = lax.fori_loop(
                0, num_experts, expert_body,
                ((neg,) * k, (zero_i,) * k))

            total = ws[0]
            for j in range(1, k):
                total = total + ws[j]
            inv = 1.0 / total
            flat = (col + lane) * k
            for j in range(k):
                plsc.store_scatter(wv, [flat + j], ws[j] * inv)
                plsc.store_scatter(iv, [flat + j], idxs[j])
            return carry

        lax.fori_loop(0, cpw // LANES, group_body, 0)
        pltpu.sync_copy(wv, w_hbm.at[pl.ds(base * k, cpw * k)])
        pltpu.sync_copy(iv, i_hbm.at[pl.ds(base * k, cpw * k)])

    return topk_sc


@jax.jit
def kernel(x, gate_w, expert_bias):
    num_tokens, dim = x.shape
    num_experts = gate_w.shape[0]
    chunk_t = num_tokens // CHUNKS
    nblk = chunk_t // BLOCK_T
    bias2d = expert_bias.reshape(num_experts, 1)

    topk = _make_topk_sc(chunk_t, num_experts, TOP_K)
    wparts, iparts = [], []
    for c in range(CHUNKS):
        scores_c = pl.pallas_call(
            _scores_body,
            grid=(nblk,),
            in_specs=[
                pl.BlockSpec((num_experts, dim), lambda i: (0, 0)),
                pl.BlockSpec((BLOCK_T, dim), lambda i, c=c: (c * nblk + i, 0)),
                pl.BlockSpec((num_experts, 1), lambda i: (0, 0)),
            ],
            out_specs=pl.BlockSpec((num_experts, BLOCK_T), lambda i: (0, i)),
            out_shape=jax.ShapeDtypeStruct((num_experts, chunk_t), jnp.float32),
        )(gate_w, x, bias2d)
        wf, if_ = topk(scores_c)
        wparts.append(wf.reshape(chunk_t, TOP_K))
        iparts.append(if_.reshape(chunk_t, TOP_K))
    return (jnp.concatenate(wparts, axis=0),
            jnp.concatenate(iparts, axis=0))


# speedup vs baseline: 1.1224x; 1.0198x over previous
import functools

import jax
import jax.numpy as jnp
from jax import lax
from jax.experimental import pallas as pl
from jax.experimental.pallas import tpu as pltpu
from jax.experimental.pallas import tpu_sc as plsc

TOP_K = 8
LANES = 16
CHUNKS = 2
BLOCK_T = 1024

_SORT8 = [(0, 1), (2, 3), (4, 5), (6, 7), (0, 2), (1, 3), (4, 6), (5, 7),
          (1, 2), (5, 6), (0, 4), (1, 5), (2, 6), (3, 7), (2, 4), (3, 5),
          (1, 2), (3, 4), (5, 6)]
_BITONIC8 = [(0, 4), (1, 5), (2, 6), (3, 7), (0, 2), (1, 3), (4, 6), (5, 7),
             (0, 1), (2, 3), (4, 5), (6, 7)]


def _scores_body(gw_ref, x_ref, b_ref, out_ref):
    logits = lax.dot_general(
        gw_ref[...], x_ref[...],
        dimension_numbers=(((1,), (1,)), ((), ())),
        preferred_element_type=jnp.float32,
    )
    out_ref[...] = jax.nn.sigmoid(logits + b_ref[...])


def _make_topk_sc(num_tokens, num_experts, k):
    info = plsc.get_sparse_core_info()
    num_workers = info.num_cores * info.num_subcores
    cpw = num_tokens // num_workers
    mesh = plsc.VectorSubcoreMesh(core_axis_name="c", subcore_axis_name="s")
    emask = num_experts - 1
    hi_mask = ~emask

    @functools.partial(
        pl.kernel,
        out_type=[
            jax.ShapeDtypeStruct((num_tokens * k,), jnp.float32),
            jax.ShapeDtypeStruct((num_tokens * k,), jnp.int32),
        ],
        mesh=mesh,
        scratch_types=[
            pltpu.VMEM((num_experts, cpw), jnp.float32),
            pltpu.VMEM((cpw * k,), jnp.float32),
            pltpu.VMEM((cpw * k,), jnp.int32),
        ],
        compiler_params=pltpu.CompilerParams(needs_layout_passes=False),
    )
    def topk_sc(scores_hbm, w_hbm, i_hbm, chunk, wv, iv):
        wid = lax.axis_index("s") * info.num_cores + lax.axis_index("c")
        base = wid * cpw
        pltpu.sync_copy(scores_hbm.at[:, pl.ds(base, cpw)], chunk)

        lane = jnp.arange(LANES, dtype=jnp.int32)

        def group_body(g, carry):
            col = g * LANES

            def sorted_block(b):
                v = [chunk[b * k + j, pl.ds(col, LANES)] for j in range(k)]
                vi = [jnp.full((LANES,), b * k + j, jnp.int32)
                      for j in range(k)]
                for (a, c) in _SORT8:
                    gt = v[c] > v[a]
                    v[a], v[c] = (jnp.where(gt, v[c], v[a]),
                                  jnp.where(gt, v[a], v[c]))
                    vi[a], vi[c] = (jnp.where(gt, vi[c], vi[a]),
                                    jnp.where(gt, vi[a], vi[c]))
                return v, vi

            ws, idxs = sorted_block(0)
            for b in range(1, num_experts // k):
                v, vi = sorted_block(b)
                nw, ni = [], []
                for i in range(k):
                    gt = v[k - 1 - i] > ws[i]
                    nw.append(jnp.where(gt, v[k - 1 - i], ws[i]))
                    ni.append(jnp.where(gt, vi[k - 1 - i], idxs[i]))
                for (a, c) in _BITONIC8:
                    gt = nw[c] > nw[a]
                    nw[a], nw[c] = (jnp.where(gt, nw[c], nw[a]),
                                    jnp.where(gt, nw[a], nw[c]))
                    ni[a], ni[c] = (jnp.where(gt, ni[c], ni[a]),
                                    jnp.where(gt, ni[a], ni[c]))
                ws, idxs = nw, ni

            total = ws[0]
            for j in range(1, k):
                total = total + ws[j]
            inv = 1.0 / total
            flat = (col + lane) * k
            for j in range(k):
                plsc.store_scatter(wv, [flat + j], ws[j] * inv)
                plsc.store_scatter(iv, [flat + j], idxs[j])
            return carry

        lax.fori_loop(0, cpw // LANES, group_body, 0)
        pltpu.sync_copy(wv, w_hbm.at[pl.ds(base * k, cpw * k)])
        pltpu.sync_copy(iv, i_hbm.at[pl.ds(base * k, cpw * k)])

    return topk_sc


@jax.jit
def kernel(x, gate_w, expert_bias):
    num_tokens, dim = x.shape
    num_experts = gate_w.shape[0]
    chunk_t = num_tokens // CHUNKS
    nblk = chunk_t // BLOCK_T
    bias2d = expert_bias.reshape(num_experts, 1)

    topk = _make_topk_sc(chunk_t, num_experts, TOP_K)
    wparts, iparts = [], []
    for c in range(CHUNKS):
        scores_c = pl.pallas_call(
            _scores_body,
            grid=(nblk,),
            in_specs=[
                pl.BlockSpec((num_experts, dim), lambda i: (0, 0)),
                pl.BlockSpec((BLOCK_T, dim), lambda i, c=c: (c * nblk + i, 0)),
                pl.BlockSpec((num_experts, 1), lambda i: (0, 0)),
            ],
            out_specs=pl.BlockSpec((num_experts, BLOCK_T), lambda i: (0, i)),
            out_shape=jax.ShapeDtypeStruct((num_experts, chunk_t), jnp.float32),
        )(gate_w, x, bias2d)
        wf, if_ = topk(scores_c)
        wparts.append(wf.reshape(chunk_t, TOP_K))
        iparts.append(if_.reshape(chunk_t, TOP_K))
    return (jnp.concatenate(wparts, axis=0),
            jnp.concatenate(iparts, axis=0))
